# baseline (device time: 122134 ns/iter reference)
import jax
import jax.numpy as jnp
from jax import lax
from jax.experimental import pallas as pl
from jax.experimental.pallas import tpu as pltpu

N_DEV = 16
SQ = 1024
D = 1024
HQ_PER = 8
DH = 128
HD = HQ_PER * DH
SCALE = 0.08838834764831843
WINDOW = 128
QB = 128
KW = 3 * QB

RS_STEPS = [(1, 512), (4, 256), (2, 128), (8, 64)]
AG_STEPS = [(8, 64), (2, 128), (4, 256), (1, 512)]
WIRE_DTYPE = jnp.bfloat16
BF = jnp.bfloat16
F32 = jnp.float32


def _fused(x, Wq_my, K, V, Wo_my):

    def body(x_ref, wq_ref, k_ref, v_ref, wo_ref, out_ref,
             p_ref, q_ref, ctx_ref, sbuf,
             r0, r1, r2, r3, a0, a1, a2, a3,
             rs_send, rs_recv, ag_send, ag_recv):
        my = lax.axis_index("i")

        barrier = pltpu.get_barrier_semaphore()
        for m in (1, 2, 4, 8):
            pl.semaphore_signal(
                barrier, inc=1,
                device_id=(jnp.bitwise_xor(my, m),),
                device_id_type=pl.DeviceIdType.MESH,
            )
        pl.semaphore_wait(barrier, 4)

        xb = x_ref[...].astype(BF)
        q_ref[...] = jnp.dot(
            xb, wq_ref[...].astype(BF), preferred_element_type=F32
        ).astype(BF)

        for i in range(SQ // QB):
            k_lo = min(max(0, QB * (i - 1)), SQ - KW)
            qi = QB * i + lax.broadcasted_iota(jnp.int32, (QB, KW), 0)
            ki = k_lo + lax.broadcasted_iota(jnp.int32, (QB, KW), 1)
            neg = jnp.where(
                jnp.abs(qi - ki) <= WINDOW, jnp.float32(0.0),
                jnp.float32(-1e9),
            )
            for h in range(HQ_PER):
                qblk = q_ref[QB * i : QB * (i + 1), DH * h : DH * (h + 1)]
                kblk = k_ref[k_lo : k_lo + KW, h, :].astype(BF)
                s = lax.dot_general(
                    qblk, kblk, (((1,), (1,)), ((), ())),
                    preferred_element_type=F32,
                ) * SCALE + neg
                mx = jnp.max(s, axis=1, keepdims=True)
                w = jnp.exp(s - mx)
                w = (w / jnp.sum(w, axis=1, keepdims=True)).astype(BF)
                vblk = v_ref[k_lo : k_lo + KW, h, :].astype(BF)
                ctx_ref[QB * i : QB * (i + 1), DH * h : DH * (h + 1)] = (
                    lax.dot_general(
                        w, vblk, (((1,), (0,)), ((), ())),
                        preferred_element_type=F32,
                    ).astype(BF)
                )

        p_ref[...] = jnp.dot(
            ctx_ref[...], wo_ref[...].astype(BF), preferred_element_type=F32
        )

        rs_rbufs = [r0, r1, r2, r3]
        ag_rbufs = [a0, a1, a2, a3]

        lo = jnp.int32(0)
        for j, (m, half) in enumerate(RS_STEPS):
            bit = jnp.bitwise_and(my // m, 1)
            keep_lo = lo + bit * half
            send_lo = lo + (1 - bit) * half
            src = p_ref if j == 0 else out_ref
            sbuf[pl.ds(0, half), :] = src[pl.ds(send_lo, half), :].astype(
                WIRE_DTYPE
            )
            rdma = pltpu.make_async_remote_copy(
                src_ref=sbuf.at[pl.ds(0, half), :],
                dst_ref=rs_rbufs[j],
                send_sem=rs_send.at[j],
                recv_sem=rs_recv.at[j],
                device_id=(jnp.bitwise_xor(my, m),),
                device_id_type=pl.DeviceIdType.MESH,
            )
            rdma.start()
            rdma.wait()
            out_ref[pl.ds(keep_lo, half), :] = (
                src[pl.ds(keep_lo, half), :]
                + rs_rbufs[j][...].astype(F32)
            )
            lo = keep_lo

        for j, (m, n) in enumerate(AG_STEPS):
            bit = jnp.bitwise_and(my // m, 1)
            merged_lo = lo - bit * n
            other_lo = merged_lo + (1 - bit) * n
            sbuf[pl.ds(0, n), :] = out_ref[pl.ds(lo, n), :].astype(WIRE_DTYPE)
            rdma = pltpu.make_async_remote_copy(
                src_ref=sbuf.at[pl.ds(0, n), :],
                dst_ref=ag_rbufs[j],
                send_sem=ag_send.at[j],
                recv_sem=ag_recv.at[j],
                device_id=(jnp.bitwise_xor(my, m),),
                device_id_type=pl.DeviceIdType.MESH,
            )
            rdma.start()
            rdma.wait()
            out_ref[pl.ds(other_lo, n), :] = ag_rbufs[j][...].astype(F32)
            lo = merged_lo

    return pl.pallas_call(
        body,
        out_shape=jax.ShapeDtypeStruct((SQ, D), F32),
        in_specs=[pl.BlockSpec(memory_space=pltpu.VMEM)] * 5,
        out_specs=pl.BlockSpec(memory_space=pltpu.VMEM),
        scratch_shapes=[
            pltpu.VMEM((SQ, D), F32),
            pltpu.VMEM((SQ, HD), BF),
            pltpu.VMEM((SQ, HD), BF),
            pltpu.VMEM((512, D), WIRE_DTYPE),
            pltpu.VMEM((512, D), WIRE_DTYPE),
            pltpu.VMEM((256, D), WIRE_DTYPE),
            pltpu.VMEM((128, D), WIRE_DTYPE),
            pltpu.VMEM((64, D), WIRE_DTYPE),
            pltpu.VMEM((64, D), WIRE_DTYPE),
            pltpu.VMEM((128, D), WIRE_DTYPE),
            pltpu.VMEM((256, D), WIRE_DTYPE),
            pltpu.VMEM((512, D), WIRE_DTYPE),
            pltpu.SemaphoreType.DMA((4,)),
            pltpu.SemaphoreType.DMA((4,)),
            pltpu.SemaphoreType.DMA((4,)),
            pltpu.SemaphoreType.DMA((4,)),
        ],
        compiler_params=pltpu.CompilerParams(collective_id=0),
    )(x, Wq_my, K, V, Wo_my)


def kernel(x, Wq, K_ext, V_ext, Wo):
    pos = lax.axis_index("i")

    Wq_my = lax.dynamic_slice(Wq, (0, pos * HD), (D, HD))
    Wo_my = lax.dynamic_slice(Wo, (pos * HD, 0), (HD, D))

    out = _fused(x[0], Wq_my, K_ext[0], V_ext[0], Wo_my)
    return out[None]


# device time: 103021 ns/iter; 1.1855x vs baseline; 1.1855x over previous
import jax
import jax.numpy as jnp
from jax import lax
from jax.experimental import pallas as pl
from jax.experimental.pallas import tpu as pltpu

N_DEV = 16
SQ = 1024
D = 1024
HQ_PER = 8
DH = 128
HD = HQ_PER * DH
SCALE = 0.08838834764831843
WINDOW = 128
QB = 128
KW = 3 * QB

RS_STEPS = [
    [(1, 256), (4, 128), (2, 64), (8, 32)],
    [(4, 256), (1, 128), (8, 64), (2, 32)],
]
AG_STEPS = [
    [(8, 32), (2, 64), (4, 128), (1, 256)],
    [(2, 32), (8, 64), (1, 128), (4, 256)],
]
PART_BASE = [0, 512]
SBUF_OFF = [0, 256]
RS_OFF = [[0, 512, 768, 896], [256, 640, 832, 928]]
AG_OFF = [[0, 64, 192, 448], [32, 128, 320, 704]]
WIRE_DTYPE = jnp.bfloat16
BF = jnp.bfloat16
F32 = jnp.float32


def _fused(x, Wq_my, K, V, Wo_my):

    def body(x_ref, wq_ref, k_ref, v_ref, wo_ref, out_ref,
             p_ref, q_ref, ctx_ref, sbuf, rsbuf, agbuf,
             rs_send, rs_recv, ag_send, ag_recv):
        my = lax.axis_index("i")

        barrier = pltpu.get_barrier_semaphore()
        for m in (1, 2, 4, 8):
            pl.semaphore_signal(
                barrier, inc=1,
                device_id=(jnp.bitwise_xor(my, m),),
                device_id_type=pl.DeviceIdType.MESH,
            )
        pl.semaphore_wait(barrier, 4)

        xb = x_ref[...].astype(BF)
        q_ref[...] = jnp.dot(
            xb, wq_ref[...].astype(BF), preferred_element_type=F32
        ).astype(BF)

        for i in range(SQ // QB):
            k_lo = min(max(0, QB * (i - 1)), SQ - KW)
            qi = QB * i + lax.broadcasted_iota(jnp.int32, (QB, KW), 0)
            ki = k_lo + lax.broadcasted_iota(jnp.int32, (QB, KW), 1)
            neg = jnp.where(
                jnp.abs(qi - ki) <= WINDOW, jnp.float32(0.0),
                jnp.float32(-1e9),
            )
            for h in range(HQ_PER):
                qblk = q_ref[QB * i : QB * (i + 1), DH * h : DH * (h + 1)]
                kblk = k_ref[k_lo : k_lo + KW, h, :].astype(BF)
                s = lax.dot_general(
                    qblk, kblk, (((1,), (1,)), ((), ())),
                    preferred_element_type=F32,
                ) * SCALE + neg
                mx = jnp.max(s, axis=1, keepdims=True)
                w = jnp.exp(s - mx)
                w = (w / jnp.sum(w, axis=1, keepdims=True)).astype(BF)
                vblk = v_ref[k_lo : k_lo + KW, h, :].astype(BF)
                ctx_ref[QB * i : QB * (i + 1), DH * h : DH * (h + 1)] = (
                    lax.dot_general(
                        w, vblk, (((1,), (0,)), ((), ())),
                        preferred_element_type=F32,
                    ).astype(BF)
                )

        p_ref[...] = jnp.dot(
            ctx_ref[...], wo_ref[...].astype(BF), preferred_element_type=F32
        )

        lo = [jnp.int32(PART_BASE[0]), jnp.int32(PART_BASE[1])]
        for j in range(4):
            src = p_ref if j == 0 else out_ref
            rdmas = []
            keeps = []
            for t in range(2):
                m, half = RS_STEPS[t][j]
                bit = jnp.bitwise_and(my // m, 1)
                keep_lo = lo[t] + bit * half
                send_lo = lo[t] + (1 - bit) * half
                sbuf[pl.ds(SBUF_OFF[t], half), :] = src[
                    pl.ds(send_lo, half), :
                ].astype(WIRE_DTYPE)
                rdma = pltpu.make_async_remote_copy(
                    src_ref=sbuf.at[pl.ds(SBUF_OFF[t], half), :],
                    dst_ref=rsbuf.at[pl.ds(RS_OFF[t][j], half), :],
                    send_sem=rs_send.at[2 * j + t],
                    recv_sem=rs_recv.at[2 * j + t],
                    device_id=(jnp.bitwise_xor(my, m),),
                    device_id_type=pl.DeviceIdType.MESH,
                )
                rdma.start()
                rdmas.append(rdma)
                keeps.append(keep_lo)
            for t in range(2):
                m, half = RS_STEPS[t][j]
                rdmas[t].wait()
                out_ref[pl.ds(keeps[t], half), :] = (
                    src[pl.ds(keeps[t], half), :]
                    + rsbuf[pl.ds(RS_OFF[t][j], half), :].astype(F32)
                )
                lo[t] = keeps[t]

        for j in range(4):
            rdmas = []
            merged = []
            others = []
            for t in range(2):
                m, n = AG_STEPS[t][j]
                bit = jnp.bitwise_and(my // m, 1)
                merged_lo = lo[t] - bit * n
                other_lo = merged_lo + (1 - bit) * n
                sbuf[pl.ds(SBUF_OFF[t], n), :] = out_ref[
                    pl.ds(lo[t], n), :
                ].astype(WIRE_DTYPE)
                rdma = pltpu.make_async_remote_copy(
                    src_ref=sbuf.at[pl.ds(SBUF_OFF[t], n), :],
                    dst_ref=agbuf.at[pl.ds(AG_OFF[t][j], n), :],
                    send_sem=ag_send.at[2 * j + t],
                    recv_sem=ag_recv.at[2 * j + t],
                    device_id=(jnp.bitwise_xor(my, m),),
                    device_id_type=pl.DeviceIdType.MESH,
                )
                rdma.start()
                rdmas.append(rdma)
                merged.append(merged_lo)
                others.append(other_lo)
            for t in range(2):
                m, n = AG_STEPS[t][j]
                rdmas[t].wait()
                out_ref[pl.ds(others[t], n), :] = agbuf[
                    pl.ds(AG_OFF[t][j], n), :
                ].astype(F32)
                lo[t] = merged[t]

    return pl.pallas_call(
        body,
        out_shape=jax.ShapeDtypeStruct((SQ, D), F32),
        in_specs=[pl.BlockSpec(memory_space=pltpu.VMEM)] * 5,
        out_specs=pl.BlockSpec(memory_space=pltpu.VMEM),
        scratch_shapes=[
            pltpu.VMEM((SQ, D), F32),
            pltpu.VMEM((SQ, HD), BF),
            pltpu.VMEM((SQ, HD), BF),
            pltpu.VMEM((512, D), WIRE_DTYPE),
            pltpu.VMEM((960, D), WIRE_DTYPE),
            pltpu.VMEM((960, D), WIRE_DTYPE),
            pltpu.SemaphoreType.DMA((8,)),
            pltpu.SemaphoreType.DMA((8,)),
            pltpu.SemaphoreType.DMA((8,)),
            pltpu.SemaphoreType.DMA((8,)),
        ],
        compiler_params=pltpu.CompilerParams(collective_id=0),
    )(x, Wq_my, K, V, Wo_my)


def kernel(x, Wq, K_ext, V_ext, Wo):
    pos = lax.axis_index("i")

    Wq_my = lax.dynamic_slice(Wq, (0, pos * HD), (D, HD))
    Wo_my = lax.dynamic_slice(Wo, (pos * HD, 0), (HD, D))

    out = _fused(x[0], Wq_my, K_ext[0], V_ext[0], Wo_my)
    return out[None]


# device time: 93154 ns/iter; 1.3111x vs baseline; 1.1059x over previous
import jax
import jax.numpy as jnp
from jax import lax
from jax.experimental import pallas as pl
from jax.experimental.pallas import tpu as pltpu

N_DEV = 16
SQ = 1024
D = 1024
HQ_PER = 8
DH = 128
HD = HQ_PER * DH
SCALE = 0.08838834764831843
WINDOW = 128
QB = 256
KW = 2 * QB

RS_STEPS = [
    [(1, 256), (4, 128), (2, 64), (8, 32)],
    [(4, 256), (1, 128), (8, 64), (2, 32)],
]
AG_STEPS = [
    [(8, 32), (2, 64), (4, 128), (1, 256)],
    [(2, 32), (8, 64), (1, 128), (4, 256)],
]
PART_BASE = [0, 512]
SBUF_OFF = [0, 256]
RS_OFF = [[0, 512, 768, 896], [256, 640, 832, 928]]
AG_OFF = [[0, 64, 192, 448], [32, 128, 320, 704]]
WIRE_DTYPE = jnp.bfloat16
BF = jnp.bfloat16
F32 = jnp.float32


def _fused(x, Wq_my, K, V, Wo_my):

    def body(x_ref, wq_ref, k_ref, v_ref, wo_ref, out_ref,
             p_ref, q_ref, ctx_ref, sbuf, rsbuf, agbuf,
             rs_send, rs_recv, ag_send, ag_recv):
        my = lax.axis_index("i")

        barrier = pltpu.get_barrier_semaphore()
        for m in (1, 2, 4, 8):
            pl.semaphore_signal(
                barrier, inc=1,
                device_id=(jnp.bitwise_xor(my, m),),
                device_id_type=pl.DeviceIdType.MESH,
            )
        pl.semaphore_wait(barrier, 4)

        xb = x_ref[...].astype(BF)
        q_ref[...] = jnp.dot(
            xb, wq_ref[...].astype(BF), preferred_element_type=F32
        ).astype(BF)

        for i in range(SQ // QB):
            k_lo = min(max(0, QB * i - WINDOW), SQ - KW)
            qi = QB * i + lax.broadcasted_iota(jnp.int32, (QB, KW), 0)
            ki = k_lo + lax.broadcasted_iota(jnp.int32, (QB, KW), 1)
            neg = jnp.where(
                jnp.abs(qi - ki) <= WINDOW, jnp.float32(0.0),
                jnp.float32(-1e9),
            )
            for h in range(HQ_PER):
                qblk = q_ref[QB * i : QB * (i + 1), DH * h : DH * (h + 1)]
                s = lax.dot_general(
                    qblk, k_ref[k_lo : k_lo + KW, h, :].astype(BF),
                    (((1,), (1,)), ((), ())),
                    preferred_element_type=F32,
                ) * SCALE + neg
                mx = jnp.max(s, axis=1, keepdims=True)
                w = jnp.exp(s - mx)
                w = (w / jnp.sum(w, axis=1, keepdims=True)).astype(BF)
                ctx_ref[QB * i : QB * (i + 1), DH * h : DH * (h + 1)] = (
                    lax.dot_general(
                        w, v_ref[k_lo : k_lo + KW, h, :].astype(BF),
                        (((1,), (0,)), ((), ())),
                        preferred_element_type=F32,
                    ).astype(BF)
                )

        p_ref[...] = jnp.dot(
            ctx_ref[...], wo_ref[...].astype(BF), preferred_element_type=F32
        )

        lo = [jnp.int32(PART_BASE[0]), jnp.int32(PART_BASE[1])]
        for j in range(4):
            src = p_ref if j == 0 else out_ref
            rdmas = []
            keeps = []
            for t in range(2):
                m, half = RS_STEPS[t][j]
                bit = jnp.bitwise_and(my // m, 1)
                keep_lo = lo[t] + bit * half
                send_lo = lo[t] + (1 - bit) * half
                sbuf[pl.ds(SBUF_OFF[t], half), :] = src[
                    pl.ds(send_lo, half), :
                ].astype(WIRE_DTYPE)
                rdma = pltpu.make_async_remote_copy(
                    src_ref=sbuf.at[pl.ds(SBUF_OFF[t], half), :],
                    dst_ref=rsbuf.at[pl.ds(RS_OFF[t][j], half), :],
                    send_sem=rs_send.at[2 * j + t],
                    recv_sem=rs_recv.at[2 * j + t],
                    device_id=(jnp.bitwise_xor(my, m),),
                    device_id_type=pl.DeviceIdType.MESH,
                )
                rdma.start()
                rdmas.append(rdma)
                keeps.append(keep_lo)
            for t in range(2):
                m, half = RS_STEPS[t][j]
                rdmas[t].wait()
                out_ref[pl.ds(keeps[t], half), :] = (
                    src[pl.ds(keeps[t], half), :]
                    + rsbuf[pl.ds(RS_OFF[t][j], half), :].astype(F32)
                )
                lo[t] = keeps[t]

        for j in range(4):
            rdmas = []
            merged = []
            others = []
            for t in range(2):
                m, n = AG_STEPS[t][j]
                bit = jnp.bitwise_and(my // m, 1)
                merged_lo = lo[t] - bit * n
                other_lo = merged_lo + (1 - bit) * n
                sbuf[pl.ds(SBUF_OFF[t], n), :] = out_ref[
                    pl.ds(lo[t], n), :
                ].astype(WIRE_DTYPE)
                rdma = pltpu.make_async_remote_copy(
                    src_ref=sbuf.at[pl.ds(SBUF_OFF[t], n), :],
                    dst_ref=agbuf.at[pl.ds(AG_OFF[t][j], n), :],
                    send_sem=ag_send.at[2 * j + t],
                    recv_sem=ag_recv.at[2 * j + t],
                    device_id=(jnp.bitwise_xor(my, m),),
                    device_id_type=pl.DeviceIdType.MESH,
                )
                rdma.start()
                rdmas.append(rdma)
                merged.append(merged_lo)
                others.append(other_lo)
            for t in range(2):
                m, n = AG_STEPS[t][j]
                rdmas[t].wait()
                out_ref[pl.ds(others[t], n), :] = agbuf[
                    pl.ds(AG_OFF[t][j], n), :
                ].astype(F32)
                lo[t] = merged[t]

    return pl.pallas_call(
        body,
        out_shape=jax.ShapeDtypeStruct((SQ, D), F32),
        in_specs=[pl.BlockSpec(memory_space=pltpu.VMEM)] * 5,
        out_specs=pl.BlockSpec(memory_space=pltpu.VMEM),
        scratch_shapes=[
            pltpu.VMEM((SQ, D), F32),
            pltpu.VMEM((SQ, HD), BF),
            pltpu.VMEM((SQ, HD), BF),
            pltpu.VMEM((512, D), WIRE_DTYPE),
            pltpu.VMEM((960, D), WIRE_DTYPE),
            pltpu.VMEM((960, D), WIRE_DTYPE),
            pltpu.SemaphoreType.DMA((8,)),
            pltpu.SemaphoreType.DMA((8,)),
            pltpu.SemaphoreType.DMA((8,)),
            pltpu.SemaphoreType.DMA((8,)),
        ],
        compiler_params=pltpu.CompilerParams(collective_id=0),
    )(x, Wq_my, K, V, Wo_my)


def kernel(x, Wq, K_ext, V_ext, Wo):
    pos = lax.axis_index("i")

    Wq_my = lax.dynamic_slice(Wq, (0, pos * HD), (D, HD))
    Wo_my = lax.dynamic_slice(Wo, (pos * HD, 0), (HD, D))

    out = _fused(x[0], Wq_my, K_ext[0], V_ext[0], Wo_my)
    return out[None]


# device time: 74751 ns/iter; 1.6339x vs baseline; 1.2462x over previous
import jax
import jax.numpy as jnp
from jax import lax
from jax.experimental import pallas as pl
from jax.experimental.pallas import tpu as pltpu

N_DEV = 16
SQ = 1024
D = 1024
HQ_PER = 8
DH = 128
HD = HQ_PER * DH
SCALE = 0.08838834764831843
WINDOW = 128
QB = 256
KW = 2 * QB

RS_STEPS = [
    [(1, 256), (4, 128), (2, 64), (8, 32)],
    [(4, 256), (1, 128), (8, 64), (2, 32)],
]
AG_STEPS = [
    [(8, 32), (2, 64), (4, 128), (1, 256)],
    [(2, 32), (8, 64), (1, 128), (4, 256)],
]
PART_BASE = [0, 512]
SBUF_OFF = [0, 256]
RS_OFF = [[0, 512, 768, 896], [256, 640, 832, 928]]
AG_OFF = [[0, 64, 192, 448], [32, 128, 320, 704]]
WIRE_DTYPE = jnp.bfloat16
BF = jnp.bfloat16
F32 = jnp.float32


def _fused(x, Wq_my, KT, V, Wo_my):

    def body(x_ref, wq_ref, kt_ref, v_ref, wo_ref, out_ref,
             p_ref, q_ref, ctx_ref, sbuf, rsbuf, agbuf,
             rs_send, rs_recv, ag_send, ag_recv):
        my = lax.axis_index("i")

        barrier = pltpu.get_barrier_semaphore()
        for m in (1, 2, 4, 8):
            pl.semaphore_signal(
                barrier, inc=1,
                device_id=(jnp.bitwise_xor(my, m),),
                device_id_type=pl.DeviceIdType.MESH,
            )
        pl.semaphore_wait(barrier, 4)

        q_ref[...] = jnp.dot(
            x_ref[...], wq_ref[...], preferred_element_type=F32
        ).astype(BF)

        for i in range(SQ // QB):
            k_lo = min(max(0, QB * i - WINDOW), SQ - KW)
            qi = QB * i + lax.broadcasted_iota(jnp.int32, (QB, KW), 0)
            ki = k_lo + lax.broadcasted_iota(jnp.int32, (QB, KW), 1)
            neg = jnp.where(
                jnp.abs(qi - ki) <= WINDOW, jnp.float32(0.0),
                jnp.float32(-1e9),
            )
            for h in range(HQ_PER):
                qblk = q_ref[QB * i : QB * (i + 1), DH * h : DH * (h + 1)]
                s = lax.dot_general(
                    qblk, kt_ref[h, :, k_lo : k_lo + KW],
                    (((1,), (0,)), ((), ())),
                    preferred_element_type=F32,
                ) * SCALE + neg
                w = jnp.exp(s)
                w = (w / jnp.sum(w, axis=1, keepdims=True)).astype(BF)
                ctx_ref[QB * i : QB * (i + 1), DH * h : DH * (h + 1)] = (
                    lax.dot_general(
                        w, v_ref[h, k_lo : k_lo + KW, :],
                        (((1,), (0,)), ((), ())),
                        preferred_element_type=F32,
                    ).astype(BF)
                )

        p_ref[...] = jnp.dot(
            ctx_ref[...], wo_ref[...], preferred_element_type=F32
        )

        lo = [jnp.int32(PART_BASE[0]), jnp.int32(PART_BASE[1])]
        for j in range(4):
            src = p_ref if j == 0 else out_ref
            rdmas = []
            keeps = []
            for t in range(2):
                m, half = RS_STEPS[t][j]
                bit = jnp.bitwise_and(my // m, 1)
                keep_lo = lo[t] + bit * half
                send_lo = lo[t] + (1 - bit) * half
                sbuf[pl.ds(SBUF_OFF[t], half), :] = src[
                    pl.ds(send_lo, half), :
                ].astype(WIRE_DTYPE)
                rdma = pltpu.make_async_remote_copy(
                    src_ref=sbuf.at[pl.ds(SBUF_OFF[t], half), :],
                    dst_ref=rsbuf.at[pl.ds(RS_OFF[t][j], half), :],
                    send_sem=rs_send.at[2 * j + t],
                    recv_sem=rs_recv.at[2 * j + t],
                    device_id=(jnp.bitwise_xor(my, m),),
                    device_id_type=pl.DeviceIdType.MESH,
                )
                rdma.start()
                rdmas.append(rdma)
                keeps.append(keep_lo)
            for t in range(2):
                m, half = RS_STEPS[t][j]
                rdmas[t].wait()
                out_ref[pl.ds(keeps[t], half), :] = (
                    src[pl.ds(keeps[t], half), :]
                    + rsbuf[pl.ds(RS_OFF[t][j], half), :].astype(F32)
                )
                lo[t] = keeps[t]

        for j in range(4):
            rdmas = []
            merged = []
            others = []
            for t in range(2):
                m, n = AG_STEPS[t][j]
                bit = jnp.bitwise_and(my // m, 1)
                merged_lo = lo[t] - bit * n
                other_lo = merged_lo + (1 - bit) * n
                sbuf[pl.ds(SBUF_OFF[t], n), :] = out_ref[
                    pl.ds(lo[t], n), :
                ].astype(WIRE_DTYPE)
                rdma = pltpu.make_async_remote_copy(
                    src_ref=sbuf.at[pl.ds(SBUF_OFF[t], n), :],
                    dst_ref=agbuf.at[pl.ds(AG_OFF[t][j], n), :],
                    send_sem=ag_send.at[2 * j + t],
                    recv_sem=ag_recv.at[2 * j + t],
                    device_id=(jnp.bitwise_xor(my, m),),
                    device_id_type=pl.DeviceIdType.MESH,
                )
                rdma.start()
                rdmas.append(rdma)
                merged.append(merged_lo)
                others.append(other_lo)
            for t in range(2):
                m, n = AG_STEPS[t][j]
                rdmas[t].wait()
                out_ref[pl.ds(others[t], n), :] = agbuf[
                    pl.ds(AG_OFF[t][j], n), :
                ].astype(F32)
                lo[t] = merged[t]

    return pl.pallas_call(
        body,
        out_shape=jax.ShapeDtypeStruct((SQ, D), F32),
        in_specs=[pl.BlockSpec(memory_space=pltpu.VMEM)] * 5,
        out_specs=pl.BlockSpec(memory_space=pltpu.VMEM),
        scratch_shapes=[
            pltpu.VMEM((SQ, D), F32),
            pltpu.VMEM((SQ, HD), BF),
            pltpu.VMEM((SQ, HD), BF),
            pltpu.VMEM((512, D), WIRE_DTYPE),
            pltpu.VMEM((960, D), WIRE_DTYPE),
            pltpu.VMEM((960, D), WIRE_DTYPE),
            pltpu.SemaphoreType.DMA((8,)),
            pltpu.SemaphoreType.DMA((8,)),
            pltpu.SemaphoreType.DMA((8,)),
            pltpu.SemaphoreType.DMA((8,)),
        ],
        compiler_params=pltpu.CompilerParams(collective_id=0),
    )(x, Wq_my, KT, V, Wo_my)


def kernel(x, Wq, K_ext, V_ext, Wo):
    pos = lax.axis_index("i")

    Wq_my = lax.dynamic_slice(Wq, (0, pos * HD), (D, HD))
    Wo_my = lax.dynamic_slice(Wo, (pos * HD, 0), (HD, D))

    xb = x[0].astype(BF)
    wqb = Wq_my.astype(BF)
    wob = Wo_my.astype(BF)
    KT = jnp.transpose(K_ext[0].astype(BF), (1, 2, 0))
    Vh = jnp.transpose(V_ext[0].astype(BF), (1, 0, 2))

    out = _fused(xb, wqb, KT, Vh, wob)
    return out[None]


# device time: 71181 ns/iter; 1.7158x vs baseline; 1.0502x over previous
import jax
import jax.numpy as jnp
from jax import lax
from jax.experimental import pallas as pl
from jax.experimental.pallas import tpu as pltpu

N_DEV = 16
SQ = 1024
D = 1024
HQ_PER = 8
DH = 128
HD = HQ_PER * DH
SCALE = 0.08838834764831843
WINDOW = 128
QB = 256
KW = 2 * QB

RS_STEPS = [
    [(1, 256), (4, 128), (2, 64), (8, 32)],
    [(4, 256), (1, 128), (8, 64), (2, 32)],
]
AG_STEPS = [
    [(8, 32), (2, 64), (4, 128), (1, 256)],
    [(2, 32), (8, 64), (1, 128), (4, 256)],
]
PART_BASE = [0, 512]
SBUF_OFF = [0, 256]
RS_OFF = [[0, 512, 768, 896], [256, 640, 832, 928]]
AG_OFF = [[0, 64, 192, 448], [32, 128, 320, 704]]
WIRE_DTYPE = jnp.bfloat16
BF = jnp.bfloat16
F32 = jnp.float32


def _fused(x, Wq_my, KT, V, Wo_my):

    def body(x_ref, wq_ref, kt_ref, v_ref, wo_ref, out_ref,
             p_ref, q_ref, ctx_ref, sbuf, rsbuf, agbuf,
             rs_send, rs_recv, ag_send, ag_recv):
        my = lax.axis_index("i")

        barrier = pltpu.get_barrier_semaphore()
        for m in (1, 2, 4, 8):
            pl.semaphore_signal(
                barrier, inc=1,
                device_id=(jnp.bitwise_xor(my, m),),
                device_id_type=pl.DeviceIdType.MESH,
            )
        pl.semaphore_wait(barrier, 4)

        q_ref[...] = (
            jnp.dot(x_ref[...], wq_ref[...], preferred_element_type=F32)
            * SCALE
        ).astype(BF)

        for i in range(SQ // QB):
            k_lo = min(max(0, QB * i - WINDOW), SQ - KW)
            qi = QB * i + lax.broadcasted_iota(jnp.int32, (QB, KW), 0)
            ki = k_lo + lax.broadcasted_iota(jnp.int32, (QB, KW), 1)
            neg = jnp.where(
                jnp.abs(qi - ki) <= WINDOW, jnp.float32(0.0),
                jnp.float32(-1e9),
            )
            for h in range(HQ_PER):
                qblk = q_ref[QB * i : QB * (i + 1), DH * h : DH * (h + 1)]
                s = lax.dot_general(
                    qblk, kt_ref[h, :, k_lo : k_lo + KW],
                    (((1,), (0,)), ((), ())),
                    preferred_element_type=F32,
                ) + neg
                w = jnp.exp(s)
                r = 1.0 / jnp.sum(w, axis=1, keepdims=True)
                ctx_ref[QB * i : QB * (i + 1), DH * h : DH * (h + 1)] = (
                    lax.dot_general(
                        w.astype(BF), v_ref[h, k_lo : k_lo + KW, :],
                        (((1,), (0,)), ((), ())),
                        preferred_element_type=F32,
                    ) * r
                ).astype(BF)

        p_ref[...] = jnp.dot(
            ctx_ref[...], wo_ref[...], preferred_element_type=F32
        )

        lo = [jnp.int32(PART_BASE[0]), jnp.int32(PART_BASE[1])]
        for j in range(4):
            src = p_ref if j == 0 else out_ref
            rdmas = []
            keeps = []
            for t in range(2):
                m, half = RS_STEPS[t][j]
                bit = jnp.bitwise_and(my // m, 1)
                keep_lo = lo[t] + bit * half
                send_lo = lo[t] + (1 - bit) * half
                sbuf[pl.ds(SBUF_OFF[t], half), :] = src[
                    pl.ds(send_lo, half), :
                ].astype(WIRE_DTYPE)
                rdma = pltpu.make_async_remote_copy(
                    src_ref=sbuf.at[pl.ds(SBUF_OFF[t], half), :],
                    dst_ref=rsbuf.at[pl.ds(RS_OFF[t][j], half), :],
                    send_sem=rs_send.at[2 * j + t],
                    recv_sem=rs_recv.at[2 * j + t],
                    device_id=(jnp.bitwise_xor(my, m),),
                    device_id_type=pl.DeviceIdType.MESH,
                )
                rdma.start()
                rdmas.append(rdma)
                keeps.append(keep_lo)
            for t in range(2):
                m, half = RS_STEPS[t][j]
                rdmas[t].wait()
                out_ref[pl.ds(keeps[t], half), :] = (
                    src[pl.ds(keeps[t], half), :]
                    + rsbuf[pl.ds(RS_OFF[t][j], half), :].astype(F32)
                )
                lo[t] = keeps[t]

        for j in range(4):
            rdmas = []
            merged = []
            others = []
            for t in range(2):
                m, n = AG_STEPS[t][j]
                bit = jnp.bitwise_and(my // m, 1)
                merged_lo = lo[t] - bit * n
                other_lo = merged_lo + (1 - bit) * n
                sbuf[pl.ds(SBUF_OFF[t], n), :] = out_ref[
                    pl.ds(lo[t], n), :
                ].astype(WIRE_DTYPE)
                rdma = pltpu.make_async_remote_copy(
                    src_ref=sbuf.at[pl.ds(SBUF_OFF[t], n), :],
                    dst_ref=agbuf.at[pl.ds(AG_OFF[t][j], n), :],
                    send_sem=ag_send.at[2 * j + t],
                    recv_sem=ag_recv.at[2 * j + t],
                    device_id=(jnp.bitwise_xor(my, m),),
                    device_id_type=pl.DeviceIdType.MESH,
                )
                rdma.start()
                rdmas.append(rdma)
                merged.append(merged_lo)
                others.append(other_lo)
            for t in range(2):
                m, n = AG_STEPS[t][j]
                rdmas[t].wait()
                out_ref[pl.ds(others[t], n), :] = agbuf[
                    pl.ds(AG_OFF[t][j], n), :
                ].astype(F32)
                lo[t] = merged[t]

    return pl.pallas_call(
        body,
        out_shape=jax.ShapeDtypeStruct((SQ, D), F32),
        in_specs=[pl.BlockSpec(memory_space=pltpu.VMEM)] * 5,
        out_specs=pl.BlockSpec(memory_space=pltpu.VMEM),
        scratch_shapes=[
            pltpu.VMEM((SQ, D), F32),
            pltpu.VMEM((SQ, HD), BF),
            pltpu.VMEM((SQ, HD), BF),
            pltpu.VMEM((512, D), WIRE_DTYPE),
            pltpu.VMEM((960, D), WIRE_DTYPE),
            pltpu.VMEM((960, D), WIRE_DTYPE),
            pltpu.SemaphoreType.DMA((8,)),
            pltpu.SemaphoreType.DMA((8,)),
            pltpu.SemaphoreType.DMA((8,)),
            pltpu.SemaphoreType.DMA((8,)),
        ],
        compiler_params=pltpu.CompilerParams(collective_id=0),
    )(x, Wq_my, KT, V, Wo_my)


def kernel(x, Wq, K_ext, V_ext, Wo):
    pos = lax.axis_index("i")

    Wq_my = lax.dynamic_slice(Wq, (0, pos * HD), (D, HD))
    Wo_my = lax.dynamic_slice(Wo, (pos * HD, 0), (HD, D))

    xb = x[0].astype(BF)
    wqb = Wq_my.astype(BF)
    wob = Wo_my.astype(BF)
    KT = jnp.transpose(K_ext[0].astype(BF), (1, 2, 0))
    Vh = jnp.transpose(V_ext[0].astype(BF), (1, 0, 2))

    out = _fused(xb, wqb, KT, Vh, wob)
    return out[None]


# device time: 69758 ns/iter; 1.7508x vs baseline; 1.0204x over previous
import jax
import jax.numpy as jnp
from jax import lax
from jax.experimental import pallas as pl
from jax.experimental.pallas import tpu as pltpu

N_DEV = 16
SQ = 1024
D = 1024
HQ_PER = 8
DH = 128
HD = HQ_PER * DH
SCALE = 0.08838834764831843
WINDOW = 128
QB = 256
KW = 2 * QB

RS_STEPS = [
    [(1, 256), (4, 128), (2, 64), (8, 32)],
    [(4, 256), (1, 128), (8, 64), (2, 32)],
]
AG_STEPS = [
    [(8, 32), (2, 64), (4, 128), (1, 256)],
    [(2, 32), (8, 64), (1, 128), (4, 256)],
]
PART_BASE = [0, 512]
SBUF_OFF = [0, 256]
RS_OFF = [[0, 512, 768, 896], [256, 640, 832, 928]]
AG_OFF = [[0, 64, 192, 448], [32, 128, 320, 704]]
WIRE_DTYPE = jnp.bfloat16
BF = jnp.bfloat16
F32 = jnp.float32


def _fused(x, Wq_my, KT, V, Wo_my):

    def body(x_ref, wq_ref, kt_ref, v_ref, wo_ref, out_ref,
             p_ref, q_ref, ctx_ref, sbuf, rsbuf, agbuf,
             rs_send, rs_recv, ag_send, ag_recv):
        my = lax.axis_index("i")

        barrier = pltpu.get_barrier_semaphore()
        for m in (1, 2, 4, 8):
            pl.semaphore_signal(
                barrier, inc=1,
                device_id=(jnp.bitwise_xor(my, m),),
                device_id_type=pl.DeviceIdType.MESH,
            )
        pl.semaphore_wait(barrier, 4)

        q_ref[...] = (
            jnp.dot(x_ref[...], wq_ref[...], preferred_element_type=F32)
            * SCALE
        ).astype(BF)

        for i in range(SQ // QB):
            k_lo = min(max(0, QB * i - WINDOW), SQ - KW)
            qi = QB * i + lax.broadcasted_iota(jnp.int32, (QB, KW), 0)
            ki = k_lo + lax.broadcasted_iota(jnp.int32, (QB, KW), 1)
            neg = jnp.where(
                jnp.abs(qi - ki) <= WINDOW, jnp.float32(0.0),
                jnp.float32(-1e9),
            )
            for h in range(HQ_PER):
                qblk = q_ref[QB * i : QB * (i + 1), DH * h : DH * (h + 1)]
                s = lax.dot_general(
                    qblk, kt_ref[h, :, k_lo : k_lo + KW],
                    (((1,), (0,)), ((), ())),
                    preferred_element_type=F32,
                ) + neg
                w = jnp.exp(s)
                r = 1.0 / jnp.sum(w, axis=1, keepdims=True)
                ctx_ref[QB * i : QB * (i + 1), DH * h : DH * (h + 1)] = (
                    lax.dot_general(
                        w.astype(BF), v_ref[h, k_lo : k_lo + KW, :],
                        (((1,), (0,)), ((), ())),
                        preferred_element_type=F32,
                    ) * r
                ).astype(BF)

        lo = [jnp.int32(PART_BASE[0]), jnp.int32(PART_BASE[1])]
        rdmas0 = []
        keeps0 = []
        for t in range(2):
            m, half = RS_STEPS[t][0]
            bit = jnp.bitwise_and(my // m, 1)
            keep_lo = lo[t] + bit * half
            send_lo = lo[t] + (1 - bit) * half
            sbuf[pl.ds(SBUF_OFF[t], half), :] = jnp.dot(
                ctx_ref[pl.ds(send_lo, half), :], wo_ref[...],
                preferred_element_type=F32,
            ).astype(WIRE_DTYPE)
            rdma = pltpu.make_async_remote_copy(
                src_ref=sbuf.at[pl.ds(SBUF_OFF[t], half), :],
                dst_ref=rsbuf.at[pl.ds(RS_OFF[t][0], half), :],
                send_sem=rs_send.at[t],
                recv_sem=rs_recv.at[t],
                device_id=(jnp.bitwise_xor(my, m),),
                device_id_type=pl.DeviceIdType.MESH,
            )
            rdma.start()
            rdmas0.append(rdma)
            keeps0.append(keep_lo)
        for t in range(2):
            _, half = RS_STEPS[t][0]
            p_ref[pl.ds(keeps0[t], half), :] = jnp.dot(
                ctx_ref[pl.ds(keeps0[t], half), :], wo_ref[...],
                preferred_element_type=F32,
            )
        for t in range(2):
            _, half = RS_STEPS[t][0]
            rdmas0[t].wait()
            out_ref[pl.ds(keeps0[t], half), :] = (
                p_ref[pl.ds(keeps0[t], half), :]
                + rsbuf[pl.ds(RS_OFF[t][0], half), :].astype(F32)
            )
            lo[t] = keeps0[t]

        for j in range(1, 4):
            src = out_ref
            rdmas = []
            keeps = []
            for t in range(2):
                m, half = RS_STEPS[t][j]
                bit = jnp.bitwise_and(my // m, 1)
                keep_lo = lo[t] + bit * half
                send_lo = lo[t] + (1 - bit) * half
                sbuf[pl.ds(SBUF_OFF[t], half), :] = src[
                    pl.ds(send_lo, half), :
                ].astype(WIRE_DTYPE)
                rdma = pltpu.make_async_remote_copy(
                    src_ref=sbuf.at[pl.ds(SBUF_OFF[t], half), :],
                    dst_ref=rsbuf.at[pl.ds(RS_OFF[t][j], half), :],
                    send_sem=rs_send.at[2 * j + t],
                    recv_sem=rs_recv.at[2 * j + t],
                    device_id=(jnp.bitwise_xor(my, m),),
                    device_id_type=pl.DeviceIdType.MESH,
                )
                rdma.start()
                rdmas.append(rdma)
                keeps.append(keep_lo)
            for t in range(2):
                m, half = RS_STEPS[t][j]
                rdmas[t].wait()
                out_ref[pl.ds(keeps[t], half), :] = (
                    src[pl.ds(keeps[t], half), :]
                    + rsbuf[pl.ds(RS_OFF[t][j], half), :].astype(F32)
                )
                lo[t] = keeps[t]

        for j in range(4):
            rdmas = []
            merged = []
            others = []
            for t in range(2):
                m, n = AG_STEPS[t][j]
                bit = jnp.bitwise_and(my // m, 1)
                merged_lo = lo[t] - bit * n
                other_lo = merged_lo + (1 - bit) * n
                sbuf[pl.ds(SBUF_OFF[t], n), :] = out_ref[
                    pl.ds(lo[t], n), :
                ].astype(WIRE_DTYPE)
                rdma = pltpu.make_async_remote_copy(
                    src_ref=sbuf.at[pl.ds(SBUF_OFF[t], n), :],
                    dst_ref=agbuf.at[pl.ds(AG_OFF[t][j], n), :],
                    send_sem=ag_send.at[2 * j + t],
                    recv_sem=ag_recv.at[2 * j + t],
                    device_id=(jnp.bitwise_xor(my, m),),
                    device_id_type=pl.DeviceIdType.MESH,
                )
                rdma.start()
                rdmas.append(rdma)
                merged.append(merged_lo)
                others.append(other_lo)
            for t in range(2):
                m, n = AG_STEPS[t][j]
                rdmas[t].wait()
                out_ref[pl.ds(others[t], n), :] = agbuf[
                    pl.ds(AG_OFF[t][j], n), :
                ].astype(F32)
                lo[t] = merged[t]

    return pl.pallas_call(
        body,
        out_shape=jax.ShapeDtypeStruct((SQ, D), F32),
        in_specs=[pl.BlockSpec(memory_space=pltpu.VMEM)] * 5,
        out_specs=pl.BlockSpec(memory_space=pltpu.VMEM),
        scratch_shapes=[
            pltpu.VMEM((SQ, D), F32),
            pltpu.VMEM((SQ, HD), BF),
            pltpu.VMEM((SQ, HD), BF),
            pltpu.VMEM((512, D), WIRE_DTYPE),
            pltpu.VMEM((960, D), WIRE_DTYPE),
            pltpu.VMEM((960, D), WIRE_DTYPE),
            pltpu.SemaphoreType.DMA((8,)),
            pltpu.SemaphoreType.DMA((8,)),
            pltpu.SemaphoreType.DMA((8,)),
            pltpu.SemaphoreType.DMA((8,)),
        ],
        compiler_params=pltpu.CompilerParams(collective_id=0),
    )(x, Wq_my, KT, V, Wo_my)


def kernel(x, Wq, K_ext, V_ext, Wo):
    pos = lax.axis_index("i")

    Wq_my = lax.dynamic_slice(Wq, (0, pos * HD), (D, HD))
    Wo_my = lax.dynamic_slice(Wo, (pos * HD, 0), (HD, D))

    xb = x[0].astype(BF)
    wqb = Wq_my.astype(BF)
    wob = Wo_my.astype(BF)
    KT = jnp.transpose(K_ext[0].astype(BF), (1, 2, 0))
    Vh = jnp.transpose(V_ext[0].astype(BF), (1, 0, 2))

    out = _fused(xb, wqb, KT, Vh, wob)
    return out[None]


# device time: 64023 ns/iter; 1.9077x vs baseline; 1.0896x over previous
import jax
import jax.numpy as jnp
from jax import lax
from jax.experimental import pallas as pl
from jax.experimental.pallas import tpu as pltpu

N_DEV = 16
SQ = 1024
D = 1024
HQ_PER = 8
DH = 128
HD = HQ_PER * DH
SCALE = 0.08838834764831843
WINDOW = 128
QB = 256
KW = 2 * QB

PLANE_G = [1, 2, 3]
Z_G = [4, 8, 12]
RS_GROUPS = [[PLANE_G, Z_G], [Z_G, PLANE_G]]
AG_GROUPS = [[Z_G, PLANE_G], [PLANE_G, Z_G]]
RS_QS = [128, 32]
AG_QS = [32, 128]
PART_BASE = [0, 512]
SOFF = [0, 384]
RS_OFF = [[0, 768], [384, 864]]
AG_OFF = [[0, 192], [96, 576]]
WIRE_DTYPE = jnp.bfloat16
BF = jnp.bfloat16
F32 = jnp.float32


def _digit(e, group):
    if group is PLANE_G:
        return jnp.bitwise_and(e, 3)
    return jnp.bitwise_and(e // 4, 3)


def _fused(x, Wq_my, KT, V, Wo_my):

    def body(x_ref, wq_ref, kt_ref, v_ref, wo_ref, out_ref,
             p_ref, q_ref, ctx_ref, sbuf, rsbuf, agbuf,
             rs_send, rs_recv, ag_send, ag_recv):
        my = lax.axis_index("i")

        barrier = pltpu.get_barrier_semaphore()
        for m in PLANE_G + Z_G:
            pl.semaphore_signal(
                barrier, inc=1,
                device_id=(jnp.bitwise_xor(my, m),),
                device_id_type=pl.DeviceIdType.MESH,
            )
        pl.semaphore_wait(barrier, 6)

        q_ref[...] = (
            jnp.dot(x_ref[...], wq_ref[...], preferred_element_type=F32)
            * SCALE
        ).astype(BF)

        for i in range(SQ // QB):
            k_lo = min(max(0, QB * i - WINDOW), SQ - KW)
            qi = QB * i + lax.broadcasted_iota(jnp.int32, (QB, KW), 0)
            ki = k_lo + lax.broadcasted_iota(jnp.int32, (QB, KW), 1)
            neg = jnp.where(
                jnp.abs(qi - ki) <= WINDOW, jnp.float32(0.0),
                jnp.float32(-1e9),
            )
            for h in range(HQ_PER):
                qblk = q_ref[QB * i : QB * (i + 1), DH * h : DH * (h + 1)]
                s = lax.dot_general(
                    qblk, kt_ref[h, :, k_lo : k_lo + KW],
                    (((1,), (0,)), ((), ())),
                    preferred_element_type=F32,
                ) + neg
                w = jnp.exp(s)
                r = 1.0 / jnp.sum(w, axis=1, keepdims=True)
                ctx_ref[QB * i : QB * (i + 1), DH * h : DH * (h + 1)] = (
                    lax.dot_general(
                        w.astype(BF), v_ref[h, k_lo : k_lo + KW, :],
                        (((1,), (0,)), ((), ())),
                        preferred_element_type=F32,
                    ) * r
                ).astype(BF)

        lo = [jnp.int32(PART_BASE[0]), jnp.int32(PART_BASE[1])]
        qs = RS_QS[0]
        rdmas0 = [[], []]
        keeps0 = []
        for t in range(2):
            grp = RS_GROUPS[t][0]
            keeps0.append(lo[t] + _digit(my, grp) * qs)
            for idx, m in enumerate(grp):
                peer = jnp.bitwise_xor(my, m)
                send_lo = lo[t] + _digit(peer, grp) * qs
                so = SOFF[t] + idx * qs
                sbuf[pl.ds(so, qs), :] = jnp.dot(
                    ctx_ref[pl.ds(send_lo, qs), :], wo_ref[...],
                    preferred_element_type=F32,
                ).astype(WIRE_DTYPE)
                rdma = pltpu.make_async_remote_copy(
                    src_ref=sbuf.at[pl.ds(so, qs), :],
                    dst_ref=rsbuf.at[pl.ds(RS_OFF[t][0] + idx * qs, qs), :],
                    send_sem=rs_send.at[3 * t + idx],
                    recv_sem=rs_recv.at[3 * t + idx],
                    device_id=(peer,),
                    device_id_type=pl.DeviceIdType.MESH,
                )
                rdma.start()
                rdmas0[t].append(rdma)
        for t in range(2):
            p_ref[pl.ds(keeps0[t], qs), :] = jnp.dot(
                ctx_ref[pl.ds(keeps0[t], qs), :], wo_ref[...],
                preferred_element_type=F32,
            )
        for t in range(2):
            for r in rdmas0[t]:
                r.wait()
            base = RS_OFF[t][0]
            out_ref[pl.ds(keeps0[t], qs), :] = (
                p_ref[pl.ds(keeps0[t], qs), :]
                + rsbuf[base : base + qs, :].astype(F32)
                + rsbuf[base + qs : base + 2 * qs, :].astype(F32)
                + rsbuf[base + 2 * qs : base + 3 * qs, :].astype(F32)
            )
            lo[t] = keeps0[t]

        qs = RS_QS[1]
        rdmas1 = [[], []]
        keeps1 = []
        for t in range(2):
            grp = RS_GROUPS[t][1]
            keeps1.append(lo[t] + _digit(my, grp) * qs)
            for idx, m in enumerate(grp):
                peer = jnp.bitwise_xor(my, m)
                send_lo = lo[t] + _digit(peer, grp) * qs
                so = SOFF[t] + idx * qs
                sbuf[pl.ds(so, qs), :] = out_ref[pl.ds(send_lo, qs), :].astype(
                    WIRE_DTYPE
                )
                rdma = pltpu.make_async_remote_copy(
                    src_ref=sbuf.at[pl.ds(so, qs), :],
                    dst_ref=rsbuf.at[pl.ds(RS_OFF[t][1] + idx * qs, qs), :],
                    send_sem=rs_send.at[6 + 3 * t + idx],
                    recv_sem=rs_recv.at[6 + 3 * t + idx],
                    device_id=(peer,),
                    device_id_type=pl.DeviceIdType.MESH,
                )
                rdma.start()
                rdmas1[t].append(rdma)
        for t in range(2):
            for r in rdmas1[t]:
                r.wait()
            base = RS_OFF[t][1]
            out_ref[pl.ds(keeps1[t], qs), :] = (
                out_ref[pl.ds(keeps1[t], qs), :]
                + rsbuf[base : base + qs, :].astype(F32)
                + rsbuf[base + qs : base + 2 * qs, :].astype(F32)
                + rsbuf[base + 2 * qs : base + 3 * qs, :].astype(F32)
            )
            lo[t] = keeps1[t]

        for j in range(2):
            n = AG_QS[j]
            rdmas = [[], []]
            bases = []
            grps = []
            for t in range(2):
                grp = AG_GROUPS[t][j]
                grps.append(grp)
                bases.append(lo[t] - _digit(my, grp) * n)
                sbuf[pl.ds(SOFF[t], n), :] = out_ref[pl.ds(lo[t], n), :].astype(
                    WIRE_DTYPE
                )
                for idx, m in enumerate(grp):
                    rdma = pltpu.make_async_remote_copy(
                        src_ref=sbuf.at[pl.ds(SOFF[t], n), :],
                        dst_ref=agbuf.at[pl.ds(AG_OFF[t][j] + idx * n, n), :],
                        send_sem=ag_send.at[6 * j + 3 * t + idx],
                        recv_sem=ag_recv.at[6 * j + 3 * t + idx],
                        device_id=(jnp.bitwise_xor(my, m),),
                        device_id_type=pl.DeviceIdType.MESH,
                    )
                    rdma.start()
                    rdmas[t].append(rdma)
            for t in range(2):
                for idx, m in enumerate(grps[t]):
                    rdmas[t][idx].wait()
                    peer = jnp.bitwise_xor(my, m)
                    dst_lo = bases[t] + _digit(peer, grps[t]) * n
                    out_ref[pl.ds(dst_lo, n), :] = agbuf[
                        pl.ds(AG_OFF[t][j] + idx * n, n), :
                    ].astype(F32)
                lo[t] = bases[t]

    return pl.pallas_call(
        body,
        out_shape=jax.ShapeDtypeStruct((SQ, D), F32),
        in_specs=[pl.BlockSpec(memory_space=pltpu.VMEM)] * 5,
        out_specs=pl.BlockSpec(memory_space=pltpu.VMEM),
        scratch_shapes=[
            pltpu.VMEM((SQ, D), F32),
            pltpu.VMEM((SQ, HD), BF),
            pltpu.VMEM((SQ, HD), BF),
            pltpu.VMEM((768, D), WIRE_DTYPE),
            pltpu.VMEM((960, D), WIRE_DTYPE),
            pltpu.VMEM((960, D), WIRE_DTYPE),
            pltpu.SemaphoreType.DMA((12,)),
            pltpu.SemaphoreType.DMA((12,)),
            pltpu.SemaphoreType.DMA((12,)),
            pltpu.SemaphoreType.DMA((12,)),
        ],
        compiler_params=pltpu.CompilerParams(collective_id=0),
    )(x, Wq_my, KT, V, Wo_my)


def kernel(x, Wq, K_ext, V_ext, Wo):
    pos = lax.axis_index("i")

    Wq_my = lax.dynamic_slice(Wq, (0, pos * HD), (D, HD))
    Wo_my = lax.dynamic_slice(Wo, (pos * HD, 0), (HD, D))

    xb = x[0].astype(BF)
    wqb = Wq_my.astype(BF)
    wob = Wo_my.astype(BF)
    KT = jnp.transpose(K_ext[0].astype(BF), (1, 2, 0))
    Vh = jnp.transpose(V_ext[0].astype(BF), (1, 0, 2))

    out = _fused(xb, wqb, KT, Vh, wob)
    return out[None]
